# Initial kernel scaffold; baseline (speedup 1.0000x reference)
import jax
import jax.numpy as jnp
from jax.experimental import pallas as pl


def kernel(x, edge_index, W1, b1, W2, b2, Wp1, bp1, Wp2, bp2, Wp3, bp3):
    raise NotImplementedError("write your pallas kernel here")


# speedup vs baseline: 2.2071x; 2.2071x over previous
import functools

import jax
import jax.numpy as jnp
from jax import lax
from jax.experimental import pallas as pl
from jax.experimental.pallas import tpu as pltpu
from jax.experimental.pallas import tpu_sc as plsc

NC = 2
NS = 16
NW = NC * NS
CH = 128


def _sc_mesh():
    return plsc.VectorSubcoreMesh(
        core_axis_name="c", subcore_axis_name="s", num_cores=NC, num_subcores=NS
    )


def _make_sc_degrees(n_pad, ecc):
    rc = n_pad // NS // CH

    @functools.partial(
        pl.kernel,
        out_type=jax.ShapeDtypeStruct((NC, n_pad, CH), jnp.float32),
        mesh=_sc_mesh(),
        scratch_types=[
            pltpu.VMEM_SHARED((n_pad, CH), jnp.float32),
            pltpu.VMEM((ecc, CH), jnp.int32),
            pltpu.VMEM((CH, CH), jnp.float32),
        ],
    )
    def k(src_h, dst_h, ones_h, zrows_h, out_h, deg, idx_v, ones_v):
        c = lax.axis_index("c")
        s = lax.axis_index("s")
        pltpu.sync_copy(zrows_h, ones_v)
        row0 = s * (rc * CH)
        for kk in range(rc):
            pltpu.sync_copy(ones_v, deg.at[pl.ds(row0 + kk * CH, CH), :])
        pltpu.sync_copy(ones_h, ones_v)

        @pl.when(c == 0)
        def _():
            pltpu.sync_copy(src_h.at[pl.ds(s * ecc, ecc), :], idx_v)

        @pl.when(c == 1)
        def _():
            pltpu.sync_copy(dst_h.at[pl.ds(s * ecc, ecc), :], idx_v)

        plsc.subcore_barrier()

        @pl.loop(0, ecc)
        def _(j):
            pltpu.sync_copy(ones_v, deg.at[idx_v.at[j]], add=True)

        plsc.subcore_barrier()
        for kk in range(rc):
            r = row0 + kk * CH
            pltpu.sync_copy(deg.at[pl.ds(r, CH), :], out_h.at[c, pl.ds(r, CH), :])

    return k


def _make_sc_aggregate(n_pad, d, ec):
    rc = n_pad // NS // CH

    @functools.partial(
        pl.kernel,
        out_type=jax.ShapeDtypeStruct((NC, n_pad, d), jnp.float32),
        mesh=_sc_mesh(),
        scratch_types=[
            pltpu.VMEM_SHARED((n_pad, d), jnp.float32),
            pltpu.VMEM((ec, CH), jnp.int32),
            pltpu.VMEM((ec, CH), jnp.int32),
            pltpu.VMEM((CH, d), jnp.float32),
            pltpu.SemaphoreType.DMA,
        ],
    )
    def k(table_h, src_h, dst_h, zrows_h, out_h, agg, src_v, dst_v, rows_v, sem):
        c = lax.axis_index("c")
        s = lax.axis_index("s")
        pltpu.sync_copy(zrows_h, rows_v)
        row0 = s * (rc * CH)
        for kk in range(rc):
            pltpu.sync_copy(rows_v, agg.at[pl.ds(row0 + kk * CH, CH), :])
        plsc.subcore_barrier()
        wid = s * NC + c
        pltpu.sync_copy(src_h.at[pl.ds(wid * ec, ec), :], src_v)
        pltpu.sync_copy(dst_h.at[pl.ds(wid * ec, ec), :], dst_v)

        @pl.loop(0, ec)
        def _(j):
            pltpu.async_copy(table_h.at[src_v.at[j]], rows_v, sem).wait()
            pltpu.sync_copy(rows_v, agg.at[dst_v.at[j]], add=True)

        plsc.subcore_barrier()
        for kk in range(rc):
            r = row0 + kk * CH
            pltpu.sync_copy(agg.at[pl.ds(r, CH), :], out_h.at[c, pl.ds(r, CH), :])

    return k


def _make_sc_edge_gather(n_pad, d, e_pad, ec):
    @functools.partial(
        pl.kernel,
        out_type=jax.ShapeDtypeStruct((e_pad, d), jnp.float32),
        mesh=_sc_mesh(),
        scratch_types=[
            pltpu.VMEM((ec, CH), jnp.int32),
            pltpu.VMEM((ec, CH), jnp.int32),
            pltpu.VMEM((CH, d), jnp.float32),
            pltpu.VMEM((CH, d), jnp.float32),
            pltpu.SemaphoreType.DMA,
            pltpu.SemaphoreType.DMA,
        ],
    )
    def k(a_h, b_h, src_h, dst_h, out_h, src_v, dst_v, a_v, b_v, sem_a, sem_b):
        c = lax.axis_index("c")
        s = lax.axis_index("s")
        wid = s * NC + c
        pltpu.sync_copy(src_h.at[pl.ds(wid * ec, ec), :], src_v)
        pltpu.sync_copy(dst_h.at[pl.ds(wid * ec, ec), :], dst_v)

        @pl.loop(0, ec)
        def _(j):
            cpa = pltpu.async_copy(a_h.at[src_v.at[j]], a_v, sem_a)
            cpb = pltpu.async_copy(b_h.at[dst_v.at[j]], b_v, sem_b)
            cpa.wait()
            cpb.wait()

            @pl.loop(0, CH)
            def _(r):
                for cc in range(d // 16):
                    sl = pl.ds(cc * 16, 16)
                    a_v[r, sl] = a_v[r, sl] + b_v[r, sl]

            pltpu.sync_copy(a_v, out_h.at[pl.ds((wid * ec + j) * CH, CH), :])

    return k


def _tc_prescale(deg_parts, x_pad, n_pad, d, rb=512):
    grid = (n_pad // rb,)

    def body(dp_r, x_r, xn_r, no_r, ni_r):
        dp = dp_r[...]
        no = lax.rsqrt(jnp.maximum(dp[0, :, 0:1], 1.0))
        ni = lax.rsqrt(jnp.maximum(dp[1, :, 0:1], 1.0))
        xn_r[...] = x_r[...] * no
        no_r[...] = no
        ni_r[...] = ni

    return pl.pallas_call(
        body,
        grid=grid,
        in_specs=[
            pl.BlockSpec((NC, rb, CH), lambda i: (0, i, 0)),
            pl.BlockSpec((rb, d), lambda i: (i, 0)),
        ],
        out_specs=[
            pl.BlockSpec((rb, d), lambda i: (i, 0)),
            pl.BlockSpec((rb, 1), lambda i: (i, 0)),
            pl.BlockSpec((rb, 1), lambda i: (i, 0)),
        ],
        out_shape=[
            jax.ShapeDtypeStruct((n_pad, d), jnp.float32),
            jax.ShapeDtypeStruct((n_pad, 1), jnp.float32),
            jax.ShapeDtypeStruct((n_pad, 1), jnp.float32),
        ],
    )(deg_parts, x_pad)


def _tc_layer1(agg_parts, ni, no, W, b, n_pad, d, rb=512):
    def body(ap_r, ni_r, no_r, w_r, b_r, out_r):
        agg = (ap_r[0] + ap_r[1]) * ni_r[...]
        h = jnp.dot(agg, w_r[...], preferred_element_type=jnp.float32) + b_r[...]
        out_r[...] = jnp.maximum(h, 0.0) * no_r[...]

    return pl.pallas_call(
        body,
        grid=(n_pad // rb,),
        in_specs=[
            pl.BlockSpec((NC, rb, d), lambda i: (0, i, 0)),
            pl.BlockSpec((rb, 1), lambda i: (i, 0)),
            pl.BlockSpec((rb, 1), lambda i: (i, 0)),
            pl.BlockSpec((d, d), lambda i: (0, 0)),
            pl.BlockSpec((1, d), lambda i: (0, 0)),
        ],
        out_specs=pl.BlockSpec((rb, d), lambda i: (i, 0)),
        out_shape=jax.ShapeDtypeStruct((n_pad, d), jnp.float32),
    )(agg_parts, ni, no, W, b)


def _tc_layer2_head(agg_parts, ni, W2, b2, Wp1, bp1, n_pad, d, rb=512):
    def body(ap_r, ni_r, w2_r, b2_r, wp1_r, bp1_r, a_r, b_out_r):
        agg = (ap_r[0] + ap_r[1]) * ni_r[...]
        h2 = jnp.dot(agg, w2_r[...], preferred_element_type=jnp.float32) + b2_r[...]
        wp1 = wp1_r[...]
        a_r[...] = jnp.dot(h2, wp1[:d], preferred_element_type=jnp.float32)
        b_out_r[...] = (
            jnp.dot(h2, wp1[d:], preferred_element_type=jnp.float32) + bp1_r[...]
        )

    return pl.pallas_call(
        body,
        grid=(n_pad // rb,),
        in_specs=[
            pl.BlockSpec((NC, rb, d), lambda i: (0, i, 0)),
            pl.BlockSpec((rb, 1), lambda i: (i, 0)),
            pl.BlockSpec((d, d), lambda i: (0, 0)),
            pl.BlockSpec((1, d), lambda i: (0, 0)),
            pl.BlockSpec((2 * d, d), lambda i: (0, 0)),
            pl.BlockSpec((1, d), lambda i: (0, 0)),
        ],
        out_specs=[
            pl.BlockSpec((rb, d), lambda i: (i, 0)),
            pl.BlockSpec((rb, d), lambda i: (i, 0)),
        ],
        out_shape=[
            jax.ShapeDtypeStruct((n_pad, d), jnp.float32),
            jax.ShapeDtypeStruct((n_pad, d), jnp.float32),
        ],
    )(agg_parts, ni, W2, b2, Wp1, bp1)


def _tc_mlp(z1pre, Wp2, bp2, Wp3, bp3, e, d, dh, rb=1280):
    def body(z_r, w2_r, b2_r, w3_r, b3_r, out_r):
        z1 = jnp.maximum(z_r[...], 0.0)
        z2 = jnp.maximum(
            jnp.dot(z1, w2_r[...], preferred_element_type=jnp.float32) + b2_r[...],
            0.0,
        )
        z3 = jnp.dot(z2, w3_r[...], preferred_element_type=jnp.float32) + b3_r[...]
        out_r[...] = jax.nn.sigmoid(z3)

    return pl.pallas_call(
        body,
        grid=(e // rb,),
        in_specs=[
            pl.BlockSpec((rb, d), lambda i: (i, 0)),
            pl.BlockSpec((d, dh), lambda i: (0, 0)),
            pl.BlockSpec((1, dh), lambda i: (0, 0)),
            pl.BlockSpec((dh, 1), lambda i: (0, 0)),
            pl.BlockSpec((1, 1), lambda i: (0, 0)),
        ],
        out_specs=pl.BlockSpec((rb, 1), lambda i: (i, 0)),
        out_shape=jax.ShapeDtypeStruct((e, 1), jnp.float32),
    )(z1pre, Wp2, bp2, Wp3, bp3)


def kernel(x, edge_index, W1, b1, W2, b2, Wp1, bp1, Wp2, bp2, Wp3, bp3):
    n, d = x.shape
    e = edge_index.shape[1]
    dh = Wp2.shape[1]

    n_pad = -(-n // (NS * CH)) * NS * CH
    e_pad = -(-e // (NW * CH * 8)) * NW * CH * 8
    ec = e_pad // NW // CH

    src = edge_index[0].astype(jnp.int32)
    dst = edge_index[1].astype(jnp.int32)
    pad = jnp.full((e_pad - e,), n, jnp.int32)
    src_h = jnp.concatenate([src, pad]).reshape(e_pad // CH, CH)
    dst_h = jnp.concatenate([dst, pad]).reshape(e_pad // CH, CH)

    x_pad = jnp.zeros((n_pad, d), jnp.float32).at[:n].set(x)
    ones_rows = jnp.ones((CH, CH), jnp.float32)
    zrows = jnp.zeros((CH, d), jnp.float32)

    ecc = e_pad // NS // CH
    sc_deg = _make_sc_degrees(n_pad, ecc)
    sc_agg = _make_sc_aggregate(n_pad, d, ec)
    sc_edge = _make_sc_edge_gather(n_pad, d, e_pad, ec)

    deg_parts = sc_deg(src_h, dst_h, ones_rows, zrows)
    xn, no, ni = _tc_prescale(deg_parts, x_pad, n_pad, d)

    agg1 = sc_agg(xn, src_h, dst_h, zrows)
    h1n = _tc_layer1(agg1, ni, no, W1, b1.reshape(1, d), n_pad, d)

    agg2 = sc_agg(h1n, src_h, dst_h, zrows)
    A, B = _tc_layer2_head(
        agg2, ni, W2, b2.reshape(1, d), Wp1, bp1.reshape(1, d), n_pad, d
    )

    z1pre = sc_edge(A, B, src_h, dst_h)
    out = _tc_mlp(z1pre, Wp2, bp2.reshape(1, dh), Wp3, bp3.reshape(1, 1), e, d, dh)
    return out


# speedup vs baseline: 2.5362x; 1.1491x over previous
import functools

import jax
import jax.numpy as jnp
from jax import lax
from jax.experimental import pallas as pl
from jax.experimental.pallas import tpu as pltpu
from jax.experimental.pallas import tpu_sc as plsc

NC = 2
NS = 16
NW = NC * NS
CH = 128


def _sc_mesh():
    return plsc.VectorSubcoreMesh(
        core_axis_name="c", subcore_axis_name="s", num_cores=NC, num_subcores=NS
    )


def _make_sc_degrees(n_pad, ecc):
    rc = n_pad // NS // CH

    @functools.partial(
        pl.kernel,
        out_type=jax.ShapeDtypeStruct((NC, n_pad, CH), jnp.float32),
        mesh=_sc_mesh(),
        scratch_types=[
            pltpu.VMEM_SHARED((n_pad, CH), jnp.float32),
            pltpu.VMEM((ecc, CH), jnp.int32),
            pltpu.VMEM((CH, CH), jnp.float32),
        ],
    )
    def k(src_h, dst_h, ones_h, zrows_h, out_h, deg, idx_v, ones_v):
        c = lax.axis_index("c")
        s = lax.axis_index("s")
        pltpu.sync_copy(zrows_h, ones_v)
        row0 = s * (rc * CH)
        for kk in range(rc):
            pltpu.sync_copy(ones_v, deg.at[pl.ds(row0 + kk * CH, CH), :])
        pltpu.sync_copy(ones_h, ones_v)

        @pl.when(c == 0)
        def _():
            pltpu.sync_copy(src_h.at[pl.ds(s * ecc, ecc), :], idx_v)

        @pl.when(c == 1)
        def _():
            pltpu.sync_copy(dst_h.at[pl.ds(s * ecc, ecc), :], idx_v)

        plsc.subcore_barrier()

        @pl.loop(0, ecc)
        def _(j):
            pltpu.sync_copy(ones_v, deg.at[idx_v.at[j]], add=True)

        plsc.subcore_barrier()
        for kk in range(rc):
            r = row0 + kk * CH
            pltpu.sync_copy(deg.at[pl.ds(r, CH), :], out_h.at[c, pl.ds(r, CH), :])

    return k


def _make_sc_aggregate(n_pad, d, ec):
    rc = n_pad // NS // CH
    w = ec // 2

    @functools.partial(
        pl.kernel,
        out_type=jax.ShapeDtypeStruct((NC, n_pad, d), jnp.float32),
        mesh=_sc_mesh(),
        scratch_types=[
            pltpu.VMEM_SHARED((n_pad, d), jnp.float32),
            pltpu.VMEM((w, CH), jnp.int32),
            pltpu.VMEM((w, CH), jnp.int32),
            pltpu.VMEM((CH, d), jnp.float32),
            pltpu.VMEM((CH, d), jnp.float32),
            pltpu.SemaphoreType.DMA,
            pltpu.SemaphoreType.DMA,
        ],
    )
    def k(table_h, src_h, dst_h, zrows_h, out_h, agg, src_v, dst_v, rows0,
          rows1, sem0, sem1):
        c = lax.axis_index("c")
        s = lax.axis_index("s")
        pltpu.sync_copy(zrows_h, rows0)
        row0 = s * (rc * CH)
        for kk in range(rc):
            pltpu.sync_copy(rows0, agg.at[pl.ds(row0 + kk * CH, CH), :])
        plsc.subcore_barrier()
        wid = s * NC + c

        for wi in range(2):
            base = wid * ec + wi * w
            pltpu.sync_copy(src_h.at[pl.ds(base, w), :], src_v)
            pltpu.sync_copy(dst_h.at[pl.ds(base, w), :], dst_v)
            pltpu.async_copy(table_h.at[src_v.at[0]], rows0, sem0)

            @pl.loop(0, w, step=2)
            def _(j):
                pltpu.async_copy(table_h.at[src_v.at[j + 1]], rows1, sem1)
                pltpu.make_async_copy(table_h.at[src_v.at[j]], rows0, sem0).wait()
                pltpu.sync_copy(rows0, agg.at[dst_v.at[j]], add=True)

                @pl.when(j + 2 < w)
                def _():
                    pltpu.async_copy(table_h.at[src_v.at[j + 2]], rows0, sem0)

                pltpu.make_async_copy(table_h.at[src_v.at[j]], rows1, sem1).wait()
                pltpu.sync_copy(rows1, agg.at[dst_v.at[j + 1]], add=True)

        plsc.subcore_barrier()
        for kk in range(rc):
            r = row0 + kk * CH
            pltpu.sync_copy(agg.at[pl.ds(r, CH), :], out_h.at[c, pl.ds(r, CH), :])

    return k


def _make_sc_edge_gather(n_pad, d, e_pad, ec):
    @functools.partial(
        pl.kernel,
        out_type=jax.ShapeDtypeStruct((e_pad, d), jnp.float32),
        mesh=_sc_mesh(),
        scratch_types=[
            pltpu.VMEM((ec, CH), jnp.int32),
            pltpu.VMEM((ec, CH), jnp.int32),
            pltpu.VMEM((CH, d), jnp.float32),
            pltpu.VMEM((CH, d), jnp.float32),
            pltpu.VMEM((CH, d), jnp.float32),
            pltpu.VMEM((CH, d), jnp.float32),
            pltpu.SemaphoreType.DMA,
            pltpu.SemaphoreType.DMA,
            pltpu.SemaphoreType.DMA,
            pltpu.SemaphoreType.DMA,
        ],
    )
    def k(a_h, b_h, src_h, dst_h, out_h, src_v, dst_v, a0, b0, a1, b1,
          sa0, sb0, sa1, sb1):
        c = lax.axis_index("c")
        s = lax.axis_index("s")
        wid = s * NC + c
        pltpu.sync_copy(src_h.at[pl.ds(wid * ec, ec), :], src_v)
        pltpu.sync_copy(dst_h.at[pl.ds(wid * ec, ec), :], dst_v)
        pltpu.async_copy(a_h.at[src_v.at[0]], a0, sa0)
        pltpu.async_copy(b_h.at[dst_v.at[0]], b0, sb0)

        def addout(av, bv, j):
            @pl.loop(0, CH)
            def _(r):
                for cc in range(d // 16):
                    sl = pl.ds(cc * 16, 16)
                    av[r, sl] = av[r, sl] + bv[r, sl]

            pltpu.sync_copy(av, out_h.at[pl.ds((wid * ec + j) * CH, CH), :])

        @pl.loop(0, ec, step=2)
        def _(j):
            pltpu.async_copy(a_h.at[src_v.at[j + 1]], a1, sa1)
            pltpu.async_copy(b_h.at[dst_v.at[j + 1]], b1, sb1)
            pltpu.make_async_copy(a_h.at[src_v.at[j]], a0, sa0).wait()
            pltpu.make_async_copy(b_h.at[dst_v.at[j]], b0, sb0).wait()
            addout(a0, b0, j)

            @pl.when(j + 2 < ec)
            def _():
                pltpu.async_copy(a_h.at[src_v.at[j + 2]], a0, sa0)
                pltpu.async_copy(b_h.at[dst_v.at[j + 2]], b0, sb0)

            pltpu.make_async_copy(a_h.at[src_v.at[j]], a1, sa1).wait()
            pltpu.make_async_copy(b_h.at[dst_v.at[j]], b1, sb1).wait()
            addout(a1, b1, j + 1)

    return k


def _tc_prescale(deg_parts, x_pad, n_pad, d, rb=512):
    grid = (n_pad // rb,)

    def body(dp_r, x_r, xn_r, no_r, ni_r):
        dp = dp_r[...]
        no = lax.rsqrt(jnp.maximum(dp[0, :, 0:1], 1.0))
        ni = lax.rsqrt(jnp.maximum(dp[1, :, 0:1], 1.0))
        xn_r[...] = x_r[...] * no
        no_r[...] = no
        ni_r[...] = ni

    return pl.pallas_call(
        body,
        grid=grid,
        in_specs=[
            pl.BlockSpec((NC, rb, CH), lambda i: (0, i, 0)),
            pl.BlockSpec((rb, d), lambda i: (i, 0)),
        ],
        out_specs=[
            pl.BlockSpec((rb, d), lambda i: (i, 0)),
            pl.BlockSpec((rb, 1), lambda i: (i, 0)),
            pl.BlockSpec((rb, 1), lambda i: (i, 0)),
        ],
        out_shape=[
            jax.ShapeDtypeStruct((n_pad, d), jnp.float32),
            jax.ShapeDtypeStruct((n_pad, 1), jnp.float32),
            jax.ShapeDtypeStruct((n_pad, 1), jnp.float32),
        ],
    )(deg_parts, x_pad)


def _tc_layer1(agg_parts, ni, no, W, b, n_pad, d, rb=512):
    def body(ap_r, ni_r, no_r, w_r, b_r, out_r):
        agg = (ap_r[0] + ap_r[1]) * ni_r[...]
        h = jnp.dot(agg, w_r[...], preferred_element_type=jnp.float32) + b_r[...]
        out_r[...] = jnp.maximum(h, 0.0) * no_r[...]

    return pl.pallas_call(
        body,
        grid=(n_pad // rb,),
        in_specs=[
            pl.BlockSpec((NC, rb, d), lambda i: (0, i, 0)),
            pl.BlockSpec((rb, 1), lambda i: (i, 0)),
            pl.BlockSpec((rb, 1), lambda i: (i, 0)),
            pl.BlockSpec((d, d), lambda i: (0, 0)),
            pl.BlockSpec((1, d), lambda i: (0, 0)),
        ],
        out_specs=pl.BlockSpec((rb, d), lambda i: (i, 0)),
        out_shape=jax.ShapeDtypeStruct((n_pad, d), jnp.float32),
    )(agg_parts, ni, no, W, b)


def _tc_layer2_head(agg_parts, ni, W2, b2, Wp1, bp1, n_pad, d, rb=512):
    def body(ap_r, ni_r, w2_r, b2_r, wp1_r, bp1_r, a_r, b_out_r):
        agg = (ap_r[0] + ap_r[1]) * ni_r[...]
        h2 = jnp.dot(agg, w2_r[...], preferred_element_type=jnp.float32) + b2_r[...]
        wp1 = wp1_r[...]
        a_r[...] = jnp.dot(h2, wp1[:d], preferred_element_type=jnp.float32)
        b_out_r[...] = (
            jnp.dot(h2, wp1[d:], preferred_element_type=jnp.float32) + bp1_r[...]
        )

    return pl.pallas_call(
        body,
        grid=(n_pad // rb,),
        in_specs=[
            pl.BlockSpec((NC, rb, d), lambda i: (0, i, 0)),
            pl.BlockSpec((rb, 1), lambda i: (i, 0)),
            pl.BlockSpec((d, d), lambda i: (0, 0)),
            pl.BlockSpec((1, d), lambda i: (0, 0)),
            pl.BlockSpec((2 * d, d), lambda i: (0, 0)),
            pl.BlockSpec((1, d), lambda i: (0, 0)),
        ],
        out_specs=[
            pl.BlockSpec((rb, d), lambda i: (i, 0)),
            pl.BlockSpec((rb, d), lambda i: (i, 0)),
        ],
        out_shape=[
            jax.ShapeDtypeStruct((n_pad, d), jnp.float32),
            jax.ShapeDtypeStruct((n_pad, d), jnp.float32),
        ],
    )(agg_parts, ni, W2, b2, Wp1, bp1)


def _tc_mlp(z1pre, Wp2, bp2, Wp3, bp3, e, d, dh, rb=1280):
    def body(z_r, w2_r, b2_r, w3_r, b3_r, out_r):
        z1 = jnp.maximum(z_r[...], 0.0)
        z2 = jnp.maximum(
            jnp.dot(z1, w2_r[...], preferred_element_type=jnp.float32) + b2_r[...],
            0.0,
        )
        z3 = jnp.dot(z2, w3_r[...], preferred_element_type=jnp.float32) + b3_r[...]
        out_r[...] = jax.nn.sigmoid(z3)

    return pl.pallas_call(
        body,
        grid=(e // rb,),
        in_specs=[
            pl.BlockSpec((rb, d), lambda i: (i, 0)),
            pl.BlockSpec((d, dh), lambda i: (0, 0)),
            pl.BlockSpec((1, dh), lambda i: (0, 0)),
            pl.BlockSpec((dh, 1), lambda i: (0, 0)),
            pl.BlockSpec((1, 1), lambda i: (0, 0)),
        ],
        out_specs=pl.BlockSpec((rb, 1), lambda i: (i, 0)),
        out_shape=jax.ShapeDtypeStruct((e, 1), jnp.float32),
    )(z1pre, Wp2, bp2, Wp3, bp3)


def kernel(x, edge_index, W1, b1, W2, b2, Wp1, bp1, Wp2, bp2, Wp3, bp3):
    n, d = x.shape
    e = edge_index.shape[1]
    dh = Wp2.shape[1]

    n_pad = -(-n // (NS * CH)) * NS * CH
    e_pad = -(-e // (NW * CH * 8)) * NW * CH * 8
    ec = e_pad // NW // CH

    src = edge_index[0].astype(jnp.int32)
    dst = edge_index[1].astype(jnp.int32)
    pad = jnp.full((e_pad - e,), n, jnp.int32)
    src_h = jnp.concatenate([src, pad]).reshape(e_pad // CH, CH)
    dst_h = jnp.concatenate([dst, pad]).reshape(e_pad // CH, CH)

    x_pad = jnp.zeros((n_pad, d), jnp.float32).at[:n].set(x)
    ones_rows = jnp.ones((CH, CH), jnp.float32)
    zrows = jnp.zeros((CH, d), jnp.float32)

    ecc = e_pad // NS // CH
    sc_deg = _make_sc_degrees(n_pad, ecc)
    sc_agg = _make_sc_aggregate(n_pad, d, ec)
    sc_edge = _make_sc_edge_gather(n_pad, d, e_pad, ec)

    deg_parts = sc_deg(src_h, dst_h, ones_rows, zrows)
    xn, no, ni = _tc_prescale(deg_parts, x_pad, n_pad, d)

    agg1 = sc_agg(xn, src_h, dst_h, zrows)
    h1n = _tc_layer1(agg1, ni, no, W1, b1.reshape(1, d), n_pad, d)

    agg2 = sc_agg(h1n, src_h, dst_h, zrows)
    A, B = _tc_layer2_head(
        agg2, ni, W2, b2.reshape(1, d), Wp1, bp1.reshape(1, d), n_pad, d
    )

    z1pre = sc_edge(A, B, src_h, dst_h)
    out = _tc_mlp(z1pre, Wp2, bp2.reshape(1, dh), Wp3, bp3.reshape(1, 1), e, d, dh)
    return out


# speedup vs baseline: 2.5544x; 1.0072x over previous
import functools

import jax
import jax.numpy as jnp
from jax import lax
from jax.experimental import pallas as pl
from jax.experimental.pallas import tpu as pltpu
from jax.experimental.pallas import tpu_sc as plsc

NC = 2
NS = 16
NW = NC * NS
CH = 128


def _sc_mesh():
    return plsc.VectorSubcoreMesh(
        core_axis_name="c", subcore_axis_name="s", num_cores=NC, num_subcores=NS
    )


def _make_sc_degrees(n_pad, ecc):
    rc = n_pad // NS // CH

    @functools.partial(
        pl.kernel,
        out_type=jax.ShapeDtypeStruct((NC, n_pad, CH), jnp.float32),
        mesh=_sc_mesh(),
        scratch_types=[
            pltpu.VMEM_SHARED((n_pad, CH), jnp.float32),
            pltpu.VMEM((ecc, CH), jnp.int32),
            pltpu.VMEM((CH, CH), jnp.float32),
        ],
    )
    def k(src_h, dst_h, ones_h, zrows_h, out_h, deg, idx_v, ones_v):
        c = lax.axis_index("c")
        s = lax.axis_index("s")
        pltpu.sync_copy(zrows_h, ones_v)
        row0 = s * (rc * CH)
        for kk in range(rc):
            pltpu.sync_copy(ones_v, deg.at[pl.ds(row0 + kk * CH, CH), :])
        pltpu.sync_copy(ones_h, ones_v)

        @pl.when(c == 0)
        def _():
            pltpu.sync_copy(src_h.at[pl.ds(s * ecc, ecc), :], idx_v)

        @pl.when(c == 1)
        def _():
            pltpu.sync_copy(dst_h.at[pl.ds(s * ecc, ecc), :], idx_v)

        plsc.subcore_barrier()

        @pl.loop(0, ecc)
        def _(j):
            pltpu.sync_copy(ones_v, deg.at[idx_v.at[j]], add=True)

        plsc.subcore_barrier()
        for kk in range(rc):
            r = row0 + kk * CH
            pltpu.sync_copy(deg.at[pl.ds(r, CH), :], out_h.at[c, pl.ds(r, CH), :])

    return k


def _make_sc_aggregate(n_pad, d, ec0, ec1):
    rc = n_pad // NS // CH
    w = ec1
    nw0 = ec0 // w

    @functools.partial(
        pl.kernel,
        out_type=jax.ShapeDtypeStruct((NC, n_pad, d), jnp.float32),
        mesh=_sc_mesh(),
        scratch_types=[
            pltpu.VMEM_SHARED((n_pad, d), jnp.float32),
            pltpu.VMEM((w, CH), jnp.int32),
            pltpu.VMEM((w, CH), jnp.int32),
            pltpu.VMEM((CH, d), jnp.float32),
            pltpu.VMEM((CH, d), jnp.float32),
            pltpu.SemaphoreType.DMA,
            pltpu.SemaphoreType.DMA,
        ],
    )
    def k(table_h, src_h, dst_h, zrows_h, out_h, agg, src_v, dst_v, rows0,
          rows1, sem0, sem1):
        c = lax.axis_index("c")
        s = lax.axis_index("s")
        pltpu.sync_copy(zrows_h, rows0)
        row0 = s * (rc * CH)
        for kk in range(rc):
            pltpu.sync_copy(rows0, agg.at[pl.ds(row0 + kk * CH, CH), :])
        plsc.subcore_barrier()
        tile_base = jnp.where(c == 0, s * ec0, NS * ec0 + s * ec1)
        nwin = jnp.where(c == 0, nw0, 1)

        @pl.loop(0, nwin)
        def _(wi):
            base = pl.multiple_of(tile_base + wi * w, 8)
            pltpu.sync_copy(src_h.at[pl.ds(base, w), :], src_v)
            pltpu.sync_copy(dst_h.at[pl.ds(base, w), :], dst_v)
            pltpu.async_copy(table_h.at[src_v.at[0]], rows0, sem0)

            @pl.loop(0, w, step=2)
            def _(j):
                pltpu.async_copy(table_h.at[src_v.at[j + 1]], rows1, sem1)
                pltpu.make_async_copy(table_h.at[src_v.at[j]], rows0, sem0).wait()
                pltpu.sync_copy(rows0, agg.at[dst_v.at[j]], add=True)

                @pl.when(j + 2 < w)
                def _():
                    pltpu.async_copy(table_h.at[src_v.at[j + 2]], rows0, sem0)

                pltpu.make_async_copy(table_h.at[src_v.at[j]], rows1, sem1).wait()
                pltpu.sync_copy(rows1, agg.at[dst_v.at[j + 1]], add=True)

        plsc.subcore_barrier()
        for kk in range(rc):
            r = row0 + kk * CH
            pltpu.sync_copy(agg.at[pl.ds(r, CH), :], out_h.at[c, pl.ds(r, CH), :])

    return k


def _make_sc_edge_gather(n_pad, d, e_pad, ec0, ec1):
    @functools.partial(
        pl.kernel,
        out_type=jax.ShapeDtypeStruct((e_pad, d), jnp.float32),
        mesh=_sc_mesh(),
        scratch_types=[
            pltpu.VMEM((ec0, CH), jnp.int32),
            pltpu.VMEM((ec0, CH), jnp.int32),
            pltpu.VMEM((CH, d), jnp.float32),
            pltpu.VMEM((CH, d), jnp.float32),
            pltpu.VMEM((CH, d), jnp.float32),
            pltpu.VMEM((CH, d), jnp.float32),
            pltpu.SemaphoreType.DMA,
            pltpu.SemaphoreType.DMA,
            pltpu.SemaphoreType.DMA,
            pltpu.SemaphoreType.DMA,
        ],
    )
    def k(a_h, b_h, src_h, dst_h, out_h, src_v, dst_v, a0, b0, a1, b1,
          sa0, sb0, sa1, sb1):
        c = lax.axis_index("c")
        s = lax.axis_index("s")
        tile_base = pl.multiple_of(
            jnp.where(c == 0, s * ec0, NS * ec0 + s * ec1), 8
        )
        ecc = jnp.where(c == 0, ec0, ec1)
        pltpu.sync_copy(src_h.at[pl.ds(tile_base, ec1), :],
                        src_v.at[pl.ds(0, ec1), :])

        @pl.when(c == 0)
        def _():
            pltpu.sync_copy(src_h.at[pl.ds(tile_base + ec1, ec0 - ec1), :],
                            src_v.at[pl.ds(ec1, ec0 - ec1), :])

        pltpu.sync_copy(dst_h.at[pl.ds(tile_base, ec1), :],
                        dst_v.at[pl.ds(0, ec1), :])

        @pl.when(c == 0)
        def _():
            pltpu.sync_copy(dst_h.at[pl.ds(tile_base + ec1, ec0 - ec1), :],
                            dst_v.at[pl.ds(ec1, ec0 - ec1), :])

        pltpu.async_copy(a_h.at[src_v.at[0]], a0, sa0)
        pltpu.async_copy(b_h.at[dst_v.at[0]], b0, sb0)

        def addout(av, bv, j):
            @pl.loop(0, CH)
            def _(r):
                for cc in range(d // 16):
                    sl = pl.ds(cc * 16, 16)
                    av[r, sl] = av[r, sl] + bv[r, sl]

            off = pl.multiple_of((tile_base + j) * CH, CH)
            pltpu.sync_copy(av, out_h.at[pl.ds(off, CH), :])

        @pl.loop(0, ecc, step=2)
        def _(j):
            pltpu.async_copy(a_h.at[src_v.at[j + 1]], a1, sa1)
            pltpu.async_copy(b_h.at[dst_v.at[j + 1]], b1, sb1)
            pltpu.make_async_copy(a_h.at[src_v.at[0]], a0, sa0).wait()
            pltpu.make_async_copy(b_h.at[dst_v.at[0]], b0, sb0).wait()
            addout(a0, b0, j)

            @pl.when(j + 2 < ecc)
            def _():
                pltpu.async_copy(a_h.at[src_v.at[j + 2]], a0, sa0)
                pltpu.async_copy(b_h.at[dst_v.at[j + 2]], b0, sb0)

            pltpu.make_async_copy(a_h.at[src_v.at[0]], a1, sa1).wait()
            pltpu.make_async_copy(b_h.at[dst_v.at[0]], b1, sb1).wait()
            addout(a1, b1, j + 1)

    return k


def _tc_prescale(deg_parts, x_pad, n_pad, d, rb=512):
    grid = (n_pad // rb,)

    def body(dp_r, x_r, xn_r, no_r, ni_r):
        dp = dp_r[...]
        no = lax.rsqrt(jnp.maximum(dp[0, :, 0:1], 1.0))
        ni = lax.rsqrt(jnp.maximum(dp[1, :, 0:1], 1.0))
        xn_r[...] = x_r[...] * no
        no_r[...] = no
        ni_r[...] = ni

    return pl.pallas_call(
        body,
        grid=grid,
        in_specs=[
            pl.BlockSpec((NC, rb, CH), lambda i: (0, i, 0)),
            pl.BlockSpec((rb, d), lambda i: (i, 0)),
        ],
        out_specs=[
            pl.BlockSpec((rb, d), lambda i: (i, 0)),
            pl.BlockSpec((rb, 1), lambda i: (i, 0)),
            pl.BlockSpec((rb, 1), lambda i: (i, 0)),
        ],
        out_shape=[
            jax.ShapeDtypeStruct((n_pad, d), jnp.float32),
            jax.ShapeDtypeStruct((n_pad, 1), jnp.float32),
            jax.ShapeDtypeStruct((n_pad, 1), jnp.float32),
        ],
    )(deg_parts, x_pad)


def _tc_layer1(agg_parts, ni, no, W, b, n_pad, d, rb=512):
    def body(ap_r, ni_r, no_r, w_r, b_r, out_r):
        agg = (ap_r[0] + ap_r[1]) * ni_r[...]
        h = jnp.dot(agg, w_r[...], preferred_element_type=jnp.float32) + b_r[...]
        out_r[...] = jnp.maximum(h, 0.0) * no_r[...]

    return pl.pallas_call(
        body,
        grid=(n_pad // rb,),
        in_specs=[
            pl.BlockSpec((NC, rb, d), lambda i: (0, i, 0)),
            pl.BlockSpec((rb, 1), lambda i: (i, 0)),
            pl.BlockSpec((rb, 1), lambda i: (i, 0)),
            pl.BlockSpec((d, d), lambda i: (0, 0)),
            pl.BlockSpec((1, d), lambda i: (0, 0)),
        ],
        out_specs=pl.BlockSpec((rb, d), lambda i: (i, 0)),
        out_shape=jax.ShapeDtypeStruct((n_pad, d), jnp.float32),
    )(agg_parts, ni, no, W, b)


def _tc_layer2_head(agg_parts, ni, W2, b2, Wp1, bp1, n_pad, d, rb=512):
    def body(ap_r, ni_r, w2_r, b2_r, wp1_r, bp1_r, a_r, b_out_r):
        agg = (ap_r[0] + ap_r[1]) * ni_r[...]
        h2 = jnp.dot(agg, w2_r[...], preferred_element_type=jnp.float32) + b2_r[...]
        wp1 = wp1_r[...]
        a_r[...] = jnp.dot(h2, wp1[:d], preferred_element_type=jnp.float32)
        b_out_r[...] = (
            jnp.dot(h2, wp1[d:], preferred_element_type=jnp.float32) + bp1_r[...]
        )

    return pl.pallas_call(
        body,
        grid=(n_pad // rb,),
        in_specs=[
            pl.BlockSpec((NC, rb, d), lambda i: (0, i, 0)),
            pl.BlockSpec((rb, 1), lambda i: (i, 0)),
            pl.BlockSpec((d, d), lambda i: (0, 0)),
            pl.BlockSpec((1, d), lambda i: (0, 0)),
            pl.BlockSpec((2 * d, d), lambda i: (0, 0)),
            pl.BlockSpec((1, d), lambda i: (0, 0)),
        ],
        out_specs=[
            pl.BlockSpec((rb, d), lambda i: (i, 0)),
            pl.BlockSpec((rb, d), lambda i: (i, 0)),
        ],
        out_shape=[
            jax.ShapeDtypeStruct((n_pad, d), jnp.float32),
            jax.ShapeDtypeStruct((n_pad, d), jnp.float32),
        ],
    )(agg_parts, ni, W2, b2, Wp1, bp1)


def _tc_mlp(z1pre, Wp2, bp2, Wp3, bp3, e, d, dh, rb=1280):
    def body(z_r, w2_r, b2_r, w3_r, b3_r, out_r):
        z1 = jnp.maximum(z_r[...], 0.0)
        z2 = jnp.maximum(
            jnp.dot(z1, w2_r[...], preferred_element_type=jnp.float32) + b2_r[...],
            0.0,
        )
        z3 = jnp.dot(z2, w3_r[...], preferred_element_type=jnp.float32) + b3_r[...]
        out_r[...] = jax.nn.sigmoid(z3)

    return pl.pallas_call(
        body,
        grid=(e // rb,),
        in_specs=[
            pl.BlockSpec((rb, d), lambda i: (i, 0)),
            pl.BlockSpec((d, dh), lambda i: (0, 0)),
            pl.BlockSpec((1, dh), lambda i: (0, 0)),
            pl.BlockSpec((dh, 1), lambda i: (0, 0)),
            pl.BlockSpec((1, 1), lambda i: (0, 0)),
        ],
        out_specs=pl.BlockSpec((rb, 1), lambda i: (i, 0)),
        out_shape=jax.ShapeDtypeStruct((e, 1), jnp.float32),
    )(z1pre, Wp2, bp2, Wp3, bp3)


def kernel(x, edge_index, W1, b1, W2, b2, Wp1, bp1, Wp2, bp2, Wp3, bp3):
    n, d = x.shape
    e = edge_index.shape[1]
    dh = Wp2.shape[1]

    n_pad = -(-n // (NS * CH)) * NS * CH
    e_pad = -(-e // (NW * CH * 8)) * NW * CH * 8
    ec = e_pad // NW // CH

    src = edge_index[0].astype(jnp.int32)
    dst = edge_index[1].astype(jnp.int32)
    pad = jnp.full((e_pad - e,), n, jnp.int32)
    src_h = jnp.concatenate([src, pad]).reshape(e_pad // CH, CH)
    dst_h = jnp.concatenate([dst, pad]).reshape(e_pad // CH, CH)
    del ec

    x_pad = jnp.zeros((n_pad, d), jnp.float32).at[:n].set(x)
    ones_rows = jnp.ones((CH, CH), jnp.float32)
    zrows = jnp.zeros((CH, d), jnp.float32)

    ecc = e_pad // NS // CH
    ec0 = (ecc * 3 // 4) // 8 * 8
    ec1 = ecc - ec0
    if ec1 <= 0 or ec0 % ec1 != 0 or ec1 % 2 != 0:
        ec0 = ec1 = ecc // 2
    sc_deg = _make_sc_degrees(n_pad, ecc)
    sc_agg = _make_sc_aggregate(n_pad, d, ec0, ec1)
    sc_edge = _make_sc_edge_gather(n_pad, d, e_pad, ec0, ec1)

    deg_parts = sc_deg(src_h, dst_h, ones_rows, zrows)
    xn, no, ni = _tc_prescale(deg_parts, x_pad, n_pad, d)

    agg1 = sc_agg(xn, src_h, dst_h, zrows)
    h1n = _tc_layer1(agg1, ni, no, W1, b1.reshape(1, d), n_pad, d)

    agg2 = sc_agg(h1n, src_h, dst_h, zrows)
    A, B = _tc_layer2_head(
        agg2, ni, W2, b2.reshape(1, d), Wp1, bp1.reshape(1, d), n_pad, d
    )

    z1pre = sc_edge(A, B, src_h, dst_h)
    out = _tc_mlp(z1pre, Wp2, bp2.reshape(1, dh), Wp3, bp3.reshape(1, 1), e, d, dh)
    return out
